# baseline (speedup 1.0000x reference)
import jax
import jax.numpy as jnp
from jax import lax
from jax.experimental import pallas as pl
from jax.experimental.pallas import tpu as pltpu
from jax.experimental.pallas import tpu_sc as plsc

BATCH = 4096
HIST = 50
DIM = 128

NC = 2
NS = 16
NW = NC * NS

HIST_PAD = 56

EL_PER_W = BATCH // NW
EL_PER_CHUNK = 2
CHUNK = EL_PER_CHUNK * HIST
N_CHUNKS = EL_PER_W // EL_PER_CHUNK
NBUF = 4
ROWBUF = 112


def _emb_kernel(idx_hbm, table_hbm, out_hbm, idx_v, rows_v, gsems, osems):
    wid = lax.axis_index("s") * NC + lax.axis_index("c")
    ebase = wid * EL_PER_W

    pltpu.sync_copy(idx_hbm.at[wid], idx_v)

    for b in range(NBUF):
        pltpu.async_copy(table_hbm.at[idx_v.at[b]],
                         rows_v.at[b, pl.ds(0, CHUNK)], gsems.at[b])

    @pl.loop(0, N_CHUNKS, step=NBUF)
    def _group(g):
        for b in range(NBUF):
            j = g + b
            pltpu.make_async_copy(table_hbm.at[idx_v.at[0]],
                                  rows_v.at[b, pl.ds(0, CHUNK)],
                                  gsems.at[b]).wait()
            e = ebase + j * EL_PER_CHUNK
            pltpu.async_copy(rows_v.at[b, pl.ds(0, HIST_PAD)],
                             out_hbm.at[e], osems.at[b])
            pltpu.async_copy(rows_v.at[b, pl.ds(HIST, HIST_PAD)],
                             out_hbm.at[e + 1], osems.at[b])

            @pl.when(j + NBUF < N_CHUNKS)
            def _():
                pltpu.make_async_copy(rows_v.at[b, pl.ds(0, HIST_PAD)],
                                      out_hbm.at[0], osems.at[b]).wait()
                pltpu.make_async_copy(rows_v.at[b, pl.ds(0, HIST_PAD)],
                                      out_hbm.at[0], osems.at[b]).wait()
                pltpu.async_copy(table_hbm.at[idx_v.at[j + NBUF]],
                                 rows_v.at[b, pl.ds(0, CHUNK)], gsems.at[b])

    for b in range(NBUF):
        pltpu.make_async_copy(rows_v.at[b, pl.ds(0, HIST_PAD)], out_hbm.at[0],
                              osems.at[b]).wait()
        pltpu.make_async_copy(rows_v.at[b, pl.ds(0, HIST_PAD)], out_hbm.at[0],
                              osems.at[b]).wait()


@jax.jit
def kernel(token_ids, weight):
    idx = token_ids.astype(jnp.int32).reshape(NW, N_CHUNKS, CHUNK)
    mesh = plsc.VectorSubcoreMesh(core_axis_name="c", subcore_axis_name="s",
                                  num_cores=NC, num_subcores=NS)
    out = pl.kernel(
        _emb_kernel,
        out_type=jax.ShapeDtypeStruct((BATCH, HIST_PAD, DIM), jnp.float32),
        mesh=mesh,
        scratch_types=[
            pltpu.VMEM((N_CHUNKS, CHUNK), jnp.int32),
            pltpu.VMEM((NBUF, ROWBUF, DIM), jnp.float32),
            pltpu.SemaphoreType.DMA((NBUF,)),
            pltpu.SemaphoreType.DMA((NBUF,)),
        ],
    )(idx, weight)
    return out[:, :HIST, :]
